# baseline (speedup 1.0000x reference)
import jax
import jax.numpy as jnp
from jax import lax
from jax.experimental import pallas as pl

NUM_CODES = 8192
CODE_DIM = 32
COMMITMENT_COST = 0.25

_BM = 256
_HALF = NUM_CODES // 2


def _half_argmin(d, base):
    m = jnp.min(d, axis=1, keepdims=True)
    io = lax.broadcasted_iota(jnp.int32, d.shape, 1)
    i = jnp.min(jnp.where(d == m, io, NUM_CODES), axis=1, keepdims=True) + base
    return i, m


def _vq_body(z_ref, cb_ref, zsq_ref, csq_ref, qst_ref, loss_ref, idx_ref):
    step = pl.program_id(0)
    zb = z_ref[...]
    cb = cb_ref[...]

    zsq = zsq_ref[...]
    csq = csq_ref[...]
    a2 = (2.0 * zb).astype(jnp.bfloat16)
    mm = lax.dot_general(a2, cb, (((1,), (1,)), ((), ())),
                         preferred_element_type=jnp.float32)
    dist = (zsq - mm) + csq

    i1, v1 = _half_argmin(dist[:, :_HALF], 0)
    i2, v2 = _half_argmin(dist[:, _HALF:], _HALF)
    u1 = lax.bitcast_convert_type(v1, jnp.int32)
    u2 = lax.bitcast_convert_type(v2, jnp.int32)
    r1 = (u1 + 0x7FFF + ((u1 >> 16) & 1)) >> 16
    r2 = u2 >> 16
    pick1 = r1 <= r2
    idx = jnp.where(pick1, i1, i2)

    iota = lax.broadcasted_iota(jnp.int32, dist.shape, 1)
    oh = (iota == idx).astype(jnp.float32)
    q = lax.dot_general(oh, cb, (((1,), (0,)), ((), ())),
                        precision=lax.Precision.HIGHEST)

    qst_ref[...] = zb + (q - zb)
    idx_ref[...] = idx

    @pl.when(step == 0)
    def _init():
        loss_ref[...] = jnp.zeros_like(loss_ref)

    loss_ref[...] += jnp.sum((zb - q) ** 2).reshape(1, 1)


def kernel(z, codebook):
    B, S, D = z.shape
    flat = z.reshape(-1, D)
    M = flat.shape[0]
    grid = M // _BM
    zsq = jnp.sum(flat ** 2, axis=1, keepdims=True)
    csq = jnp.sum(codebook ** 2, axis=1)[None, :]

    qst, loss, idx = pl.pallas_call(
        _vq_body,
        grid=(grid,),
        in_specs=[
            pl.BlockSpec((_BM, D), lambda i: (i, 0)),
            pl.BlockSpec((NUM_CODES, D), lambda i: (0, 0)),
            pl.BlockSpec((_BM, 1), lambda i: (i, 0)),
            pl.BlockSpec((1, NUM_CODES), lambda i: (0, 0)),
        ],
        out_specs=[
            pl.BlockSpec((_BM, D), lambda i: (i, 0)),
            pl.BlockSpec((1, 1), lambda i: (0, 0)),
            pl.BlockSpec((_BM, 1), lambda i: (i, 0)),
        ],
        out_shape=[
            jax.ShapeDtypeStruct((M, D), jnp.float32),
            jax.ShapeDtypeStruct((1, 1), jnp.float32),
            jax.ShapeDtypeStruct((M, 1), jnp.int32),
        ],
    )(flat, codebook, zsq, csq)

    mean_loss = loss[0, 0] / (M * D)
    vq_loss = mean_loss + COMMITMENT_COST * mean_loss
    return qst.reshape(z.shape), vq_loss, idx.reshape(B, S)


# speedup vs baseline: 2.6835x; 2.6835x over previous
import jax
import jax.numpy as jnp
from jax import lax
from jax.experimental import pallas as pl

NUM_CODES = 8192
CODE_DIM = 32
COMMITMENT_COST = 0.25

_BM = 256
_HALF = NUM_CODES // 2


def _half_argmin(d, base):
    m = jnp.min(d, axis=1, keepdims=True)
    io = lax.broadcasted_iota(jnp.int32, d.shape, 1)
    i = jnp.min(jnp.where(d == m, io, NUM_CODES), axis=1, keepdims=True) + base
    return i, m


def _vq_body(z_ref, cb_ref, zsq_ref, csq_ref, qst_ref, loss_ref, idx_ref):
    step = pl.program_id(0)
    zb = z_ref[...]
    cb = cb_ref[...]

    zsq = zsq_ref[...]
    csq = csq_ref[...]
    a2 = (2.0 * zb).astype(jnp.bfloat16)
    mm = lax.dot_general(a2, cb, (((1,), (1,)), ((), ())),
                         preferred_element_type=jnp.float32)
    dist = (zsq - mm) + csq

    i1, v1 = _half_argmin(dist[:, :_HALF], 0)
    i2, v2 = _half_argmin(dist[:, _HALF:], _HALF)
    u1 = lax.bitcast_convert_type(v1, jnp.int32)
    u2 = lax.bitcast_convert_type(v2, jnp.int32)
    r1 = (u1 + 0x7FFF + ((u1 >> 16) & 1)) >> 16
    r2 = u2 >> 16
    pick1 = r1 <= r2
    idx = jnp.where(pick1, i1, i2)

    iota = lax.broadcasted_iota(jnp.int32, dist.shape, 1)
    oh = (iota == idx).astype(jnp.bfloat16)
    q = lax.dot_general(oh, cb, (((1,), (0,)), ((), ())),
                        preferred_element_type=jnp.float32)

    qst_ref[...] = zb + (q - zb)
    idx_ref[...] = idx

    @pl.when(step == 0)
    def _init():
        loss_ref[...] = jnp.zeros_like(loss_ref)

    loss_ref[...] += jnp.sum((zb - q) ** 2).reshape(1, 1)


def kernel(z, codebook):
    B, S, D = z.shape
    flat = z.reshape(-1, D)
    M = flat.shape[0]
    grid = M // _BM
    zsq = jnp.sum(flat ** 2, axis=1, keepdims=True)
    csq = jnp.sum(codebook ** 2, axis=1)[None, :]

    qst, loss, idx = pl.pallas_call(
        _vq_body,
        grid=(grid,),
        in_specs=[
            pl.BlockSpec((_BM, D), lambda i: (i, 0)),
            pl.BlockSpec((NUM_CODES, D), lambda i: (0, 0)),
            pl.BlockSpec((_BM, 1), lambda i: (i, 0)),
            pl.BlockSpec((1, NUM_CODES), lambda i: (0, 0)),
        ],
        out_specs=[
            pl.BlockSpec((_BM, D), lambda i: (i, 0)),
            pl.BlockSpec((1, 1), lambda i: (0, 0)),
            pl.BlockSpec((_BM, 1), lambda i: (i, 0)),
        ],
        out_shape=[
            jax.ShapeDtypeStruct((M, D), jnp.float32),
            jax.ShapeDtypeStruct((1, 1), jnp.float32),
            jax.ShapeDtypeStruct((M, 1), jnp.int32),
        ],
    )(flat, codebook, zsq, csq)

    mean_loss = loss[0, 0] / (M * D)
    vq_loss = mean_loss + COMMITMENT_COST * mean_loss
    return qst.reshape(z.shape), vq_loss, idx.reshape(B, S)
